# baseline (device time: 356263 ns/iter reference)
import jax
import jax.numpy as jnp
from jax import lax
from jax.experimental import pallas as pl
from jax.experimental.pallas import tpu as pltpu

N_DEV = 32
N_STEPS = 5
B, S, D = 2, 256, 1024
H, Dh, Dr = 16, 64, 32
ROWS = B * S
KV_COLS = 2 * D


def _allreduce_kv_body(x_ref, wdkv_ref, wuk_ref, wuv_ref, kv_ref,
                       slots_ref, send_sems, recv_sems):
    my = lax.axis_index("i")

    x2 = x_ref[...].reshape(ROWS, D)
    c = jnp.dot(x2, wdkv_ref[...], preferred_element_type=jnp.float32)
    kv_ref[:, 0:D] = jnp.dot(c, wuk_ref[...],
                             preferred_element_type=jnp.float32)
    kv_ref[:, D:KV_COLS] = jnp.dot(c, wuv_ref[...],
                                   preferred_element_type=jnp.float32)

    barrier_sem = pltpu.get_barrier_semaphore()
    for step in range(N_STEPS):
        partner = my ^ (1 << step)
        pl.semaphore_signal(barrier_sem, inc=1, device_id=(partner,),
                            device_id_type=pl.DeviceIdType.MESH)
    pl.semaphore_wait(barrier_sem, N_STEPS)

    for step in range(N_STEPS):
        partner = my ^ (1 << step)
        rdma = pltpu.make_async_remote_copy(
            src_ref=kv_ref,
            dst_ref=slots_ref.at[step],
            send_sem=send_sems.at[step],
            recv_sem=recv_sems.at[step],
            device_id=(partner,),
            device_id_type=pl.DeviceIdType.MESH,
        )
        rdma.start()
        rdma.wait()
        kv_ref[...] = kv_ref[...] + slots_ref[step]


def _allreduce_kv(x, Wdkv, Wuk, Wuv):
    return pl.pallas_call(
        _allreduce_kv_body,
        out_shape=jax.ShapeDtypeStruct((ROWS, KV_COLS), jnp.float32),
        in_specs=[pl.BlockSpec(memory_space=pltpu.VMEM)] * 4,
        out_specs=pl.BlockSpec(memory_space=pltpu.VMEM),
        scratch_shapes=[
            pltpu.VMEM((N_STEPS, ROWS, KV_COLS), jnp.float32),
            pltpu.SemaphoreType.DMA((N_STEPS,)),
            pltpu.SemaphoreType.DMA((N_STEPS,)),
        ],
        compiler_params=pltpu.CompilerParams(collective_id=0),
    )(x, Wdkv, Wuk, Wuv)


def kernel(x, Wdkv, Wuk, Wuv, Wq, Wqr, Wkr, Wo):
    kv = _allreduce_kv(x, Wdkv, Wuk, Wuv)
    K = kv[:, :D].reshape(B, S, H, Dh)
    V = kv[:, D:].reshape(B, S, H, Dh)
    x2 = x.reshape(ROWS, D)
    Q = (x2 @ Wq).reshape(B, S, H, Dh)
    Qr = (x2 @ Wqr).reshape(B, S, H, Dr)
    Kr = (x2 @ Wkr).reshape(B, S, Dr)
    scale = (Dh + Dr) ** -0.5
    scores = (jnp.einsum("bshd,bthd->bhst", Q, K)
              + jnp.einsum("bshd,btd->bhst", Qr, Kr)) * scale
    m = scores.max(-1, keepdims=True)
    p = jnp.exp(scores - m)
    p = p / p.sum(-1, keepdims=True)
    O = jnp.einsum("bhst,bthd->bshd", p, V).reshape(ROWS, H * Dh)
    return (O @ Wo).reshape(B, S, D)


# device time: 146881 ns/iter; 2.4255x vs baseline; 2.4255x over previous
import jax
import jax.numpy as jnp
from jax import lax
from jax.experimental import pallas as pl
from jax.experimental.pallas import tpu as pltpu

N_DEV = 32
N_STEPS = 5
B, S, D = 2, 256, 1024
H, Dh, Dr = 16, 64, 32
ROWS = B * S
KV_COLS = 2 * D


def _allreduce_kv_body(x_ref, wdkv_ref, wuk_ref, wuv_ref, kv_ref,
                       s0, s1, s2, s3, s4,
                       a0, a1, a2, a3, a4,
                       t0, t1, t2, t3, t4,
                       rs_send, rs_recv, ag_send, ag_recv):
    my = lax.axis_index("i")
    slots = [s0, s1, s2, s3, s4]
    ag_slots = [a0, a1, a2, a3, a4]
    stage = [t0, t1, t2, t3, t4]

    x2 = x_ref[...].reshape(ROWS, D)
    c = jnp.dot(x2, wdkv_ref[...], preferred_element_type=jnp.float32)
    kv_ref[:, 0:D] = jnp.dot(c, wuk_ref[...],
                             preferred_element_type=jnp.float32)
    kv_ref[:, D:KV_COLS] = jnp.dot(c, wuv_ref[...],
                                   preferred_element_type=jnp.float32)

    barrier_sem = pltpu.get_barrier_semaphore()
    for step in range(N_STEPS):
        partner = my ^ (1 << step)
        pl.semaphore_signal(barrier_sem, inc=1, device_id=(partner,),
                            device_id_type=pl.DeviceIdType.MESH)
    pl.semaphore_wait(barrier_sem, N_STEPS)

    lo = jnp.int32(0)
    n = ROWS
    for step in range(N_STEPS):
        partner = my ^ (1 << step)
        bit = (my >> step) & 1
        half = n // 2
        keep_lo = lo + bit * half
        send_lo = lo + (1 - bit) * half
        stage[step][...] = kv_ref[pl.ds(send_lo, half), :]
        rdma = pltpu.make_async_remote_copy(
            src_ref=stage[step],
            dst_ref=slots[step],
            send_sem=rs_send.at[step],
            recv_sem=rs_recv.at[step],
            device_id=(partner,),
            device_id_type=pl.DeviceIdType.MESH,
        )
        rdma.start()
        rdma.wait()
        kv_ref[pl.ds(keep_lo, half), :] = (
            kv_ref[pl.ds(keep_lo, half), :] + slots[step][...]
        )
        lo = keep_lo
        n = half

    for step in reversed(range(N_STEPS)):
        partner = my ^ (1 << step)
        bit = (my >> step) & 1
        parent_lo = lo - bit * n
        partner_lo = parent_lo + (1 - bit) * n
        stage[step][...] = kv_ref[pl.ds(lo, n), :]
        rdma = pltpu.make_async_remote_copy(
            src_ref=stage[step],
            dst_ref=ag_slots[step],
            send_sem=ag_send.at[step],
            recv_sem=ag_recv.at[step],
            device_id=(partner,),
            device_id_type=pl.DeviceIdType.MESH,
        )
        rdma.start()
        rdma.wait()
        kv_ref[pl.ds(partner_lo, n), :] = ag_slots[step][...]
        lo = parent_lo
        n = 2 * n


def _allreduce_kv(x, Wdkv, Wuk, Wuv):
    return pl.pallas_call(
        _allreduce_kv_body,
        out_shape=jax.ShapeDtypeStruct((ROWS, KV_COLS), jnp.float32),
        in_specs=[pl.BlockSpec(memory_space=pltpu.VMEM)] * 4,
        out_specs=pl.BlockSpec(memory_space=pltpu.VMEM),
        scratch_shapes=(
            [pltpu.VMEM((ROWS >> (s + 1), KV_COLS), jnp.float32)
             for s in range(N_STEPS)] * 3
            + [pltpu.SemaphoreType.DMA((N_STEPS,))] * 4
        ),
        compiler_params=pltpu.CompilerParams(collective_id=0),
    )(x, Wdkv, Wuk, Wuv)


def kernel(x, Wdkv, Wuk, Wuv, Wq, Wqr, Wkr, Wo):
    kv = _allreduce_kv(x, Wdkv, Wuk, Wuv)
    K = kv[:, :D].reshape(B, S, H, Dh)
    V = kv[:, D:].reshape(B, S, H, Dh)
    x2 = x.reshape(ROWS, D)
    Q = (x2 @ Wq).reshape(B, S, H, Dh)
    Qr = (x2 @ Wqr).reshape(B, S, H, Dr)
    Kr = (x2 @ Wkr).reshape(B, S, Dr)
    scale = (Dh + Dr) ** -0.5
    scores = (jnp.einsum("bshd,bthd->bhst", Q, K)
              + jnp.einsum("bshd,btd->bhst", Qr, Kr)) * scale
    m = scores.max(-1, keepdims=True)
    p = jnp.exp(scores - m)
    p = p / p.sum(-1, keepdims=True)
    O = jnp.einsum("bhst,bthd->bshd", p, V).reshape(ROWS, H * Dh)
    return (O @ Wo).reshape(B, S, D)


# device time: 96507 ns/iter; 3.6916x vs baseline; 1.5220x over previous
import jax
import jax.numpy as jnp
from jax import lax
from jax.experimental import pallas as pl
from jax.experimental.pallas import tpu as pltpu

N_DEV = 32
N_STEPS = 5
B, S, D = 2, 256, 1024
H, Dh, Dr = 16, 64, 32
ROWS = B * S
KV_COLS = 2 * D
N_STREAMS = 2
SROWS = ROWS // N_STREAMS

_AXES = {
    "x": (1, 0, 0),
    "y1": (0, 1, 0),
    "y2": (0, 2, 0),
    "z1": (0, 0, 1),
    "z2": (0, 0, 2),
}
_ORDERS = [
    ["x", "y1", "z1", "z2", "y2"],
    ["y1", "z1", "x", "y2", "z2"],
]


def _coords(my):
    z = my >> 3
    q = my & 7
    y = q >> 1
    x = (q & 1) ^ (y & 1)
    return x, y, z


def _partner_and_bit(my, axis):
    fx, fy, fz = _AXES[axis]
    x, y, z = _coords(my)
    px, py, pz = x ^ fx, y ^ fy, z ^ fz
    ppos = (pz << 3) + (py << 1) + ((px ^ (py & 1)) & 1)
    if axis == "x":
        bit = x
    elif axis == "y1":
        bit = y & 1
    elif axis == "y2":
        bit = (y >> 1) & 1
    elif axis == "z1":
        bit = z & 1
    else:
        bit = (z >> 1) & 1
    return ppos, bit


def _allreduce_kv_body(x_ref, wdkv_ref, wuk_ref, wuv_ref, kv_ref,
                       *scratch):
    slots = scratch[0:N_STEPS]
    ag_slots = scratch[N_STEPS:2 * N_STEPS]
    stage = scratch[2 * N_STEPS:3 * N_STEPS]
    rs_send, rs_recv, ag_send, ag_recv = scratch[3 * N_STEPS:]
    my = lax.axis_index("i")

    x2 = x_ref[...].reshape(ROWS, D)
    c = jnp.dot(x2, wdkv_ref[...], preferred_element_type=jnp.float32)
    kv_ref[:, 0:D] = jnp.dot(c, wuk_ref[...],
                             preferred_element_type=jnp.float32)
    kv_ref[:, D:KV_COLS] = jnp.dot(c, wuv_ref[...],
                                   preferred_element_type=jnp.float32)

    barrier_sem = pltpu.get_barrier_semaphore()
    for axis in _AXES:
        partner, _ = _partner_and_bit(my, axis)
        pl.semaphore_signal(barrier_sem, inc=1, device_id=(partner,),
                            device_id_type=pl.DeviceIdType.MESH)
    pl.semaphore_wait(barrier_sem, N_STEPS)

    lo = [jnp.int32(st * SROWS) for st in range(N_STREAMS)]
    n = [SROWS] * N_STREAMS
    for step in range(N_STEPS):
        rdmas = []
        for st in range(N_STREAMS):
            partner, bit = _partner_and_bit(my, _ORDERS[st][step])
            half = n[st] // 2
            keep_lo = lo[st] + bit * half
            send_lo = lo[st] + (1 - bit) * half
            stage[step][st] = kv_ref[pl.ds(send_lo, half), :]
            rdma = pltpu.make_async_remote_copy(
                src_ref=stage[step].at[st],
                dst_ref=slots[step].at[st],
                send_sem=rs_send.at[step, st],
                recv_sem=rs_recv.at[step, st],
                device_id=(partner,),
                device_id_type=pl.DeviceIdType.MESH,
            )
            rdma.start()
            rdmas.append(rdma)
            lo[st] = keep_lo
            n[st] = half
        for st in range(N_STREAMS):
            rdmas[st].wait()
            kv_ref[pl.ds(lo[st], n[st]), :] = (
                kv_ref[pl.ds(lo[st], n[st]), :] + slots[step][st]
            )

    for step in reversed(range(N_STEPS)):
        rdmas = []
        plos = []
        for st in range(N_STREAMS):
            partner, bit = _partner_and_bit(my, _ORDERS[st][step])
            parent_lo = lo[st] - bit * n[st]
            plos.append(parent_lo + (1 - bit) * n[st])
            stage[step][st] = kv_ref[pl.ds(lo[st], n[st]), :]
            rdma = pltpu.make_async_remote_copy(
                src_ref=stage[step].at[st],
                dst_ref=ag_slots[step].at[st],
                send_sem=ag_send.at[step, st],
                recv_sem=ag_recv.at[step, st],
                device_id=(partner,),
                device_id_type=pl.DeviceIdType.MESH,
            )
            rdma.start()
            rdmas.append(rdma)
            lo[st] = parent_lo
            n[st] = 2 * n[st]
        for st in range(N_STREAMS):
            rdmas[st].wait()
            kv_ref[pl.ds(plos[st], n[st] // 2), :] = ag_slots[step][st]


def _allreduce_kv(x, Wdkv, Wuk, Wuv):
    return pl.pallas_call(
        _allreduce_kv_body,
        out_shape=jax.ShapeDtypeStruct((ROWS, KV_COLS), jnp.float32),
        in_specs=[pl.BlockSpec(memory_space=pltpu.VMEM)] * 4,
        out_specs=pl.BlockSpec(memory_space=pltpu.VMEM),
        scratch_shapes=(
            [pltpu.VMEM((N_STREAMS, SROWS >> (s + 1), KV_COLS), jnp.float32)
             for s in range(N_STEPS)] * 3
            + [pltpu.SemaphoreType.DMA((N_STEPS, N_STREAMS))] * 4
        ),
        compiler_params=pltpu.CompilerParams(collective_id=0),
    )(x, Wdkv, Wuk, Wuv)


def kernel(x, Wdkv, Wuk, Wuv, Wq, Wqr, Wkr, Wo):
    kv = _allreduce_kv(x, Wdkv, Wuk, Wuv)
    K = kv[:, :D].reshape(B, S, H, Dh)
    V = kv[:, D:].reshape(B, S, H, Dh)
    x2 = x.reshape(ROWS, D)
    Q = (x2 @ Wq).reshape(B, S, H, Dh)
    Qr = (x2 @ Wqr).reshape(B, S, H, Dr)
    Kr = (x2 @ Wkr).reshape(B, S, Dr)
    scale = (Dh + Dr) ** -0.5
    scores = (jnp.einsum("bshd,bthd->bhst", Q, K)
              + jnp.einsum("bshd,btd->bhst", Qr, Kr)) * scale
    m = scores.max(-1, keepdims=True)
    p = jnp.exp(scores - m)
    p = p / p.sum(-1, keepdims=True)
    O = jnp.einsum("bhst,bthd->bshd", p, V).reshape(ROWS, H * Dh)
    return (O @ Wo).reshape(B, S, D)


# device time: 95920 ns/iter; 3.7142x vs baseline; 1.0061x over previous
import jax
import jax.numpy as jnp
from jax import lax
from jax.experimental import pallas as pl
from jax.experimental.pallas import tpu as pltpu

N_DEV = 32
N_STEPS = 5
B, S, D = 2, 256, 1024
H, Dh, Dr = 16, 64, 32
ROWS = B * S
KV_COLS = 2 * D
N_STREAMS = 2
SROWS = ROWS // N_STREAMS

_AXES = {
    "x": (1, 0, 0),
    "y1": (0, 1, 0),
    "y2": (0, 2, 0),
    "z1": (0, 0, 1),
    "z2": (0, 0, 2),
}
_ORDERS = [
    ["x", "y1", "z1", "z2", "y2"],
    ["y1", "z1", "x", "y2", "z2"],
]


def _coords(my):
    z = my >> 3
    q = my & 7
    y = q >> 1
    x = (q & 1) ^ (y & 1)
    return x, y, z


def _partner_and_bit(my, axis):
    fx, fy, fz = _AXES[axis]
    x, y, z = _coords(my)
    px, py, pz = x ^ fx, y ^ fy, z ^ fz
    ppos = (pz << 3) + (py << 1) + ((px ^ (py & 1)) & 1)
    if axis == "x":
        bit = x
    elif axis == "y1":
        bit = y & 1
    elif axis == "y2":
        bit = (y >> 1) & 1
    elif axis == "z1":
        bit = z & 1
    else:
        bit = (z >> 1) & 1
    return ppos, bit


def _allreduce_kv_body(x_ref, wdkv_ref, wuk_ref, wuv_ref, kv_ref,
                       *scratch):
    slots = scratch[0:N_STEPS]
    rs_send, rs_recv, ag_send, ag_recv = scratch[N_STEPS:]
    my = lax.axis_index("i")

    x2 = x_ref[...].reshape(ROWS, D)
    c = jnp.dot(x2, wdkv_ref[...], preferred_element_type=jnp.float32)
    kv_ref[:, 0:D] = jnp.dot(c, wuk_ref[...],
                             preferred_element_type=jnp.float32)
    kv_ref[:, D:KV_COLS] = jnp.dot(c, wuv_ref[...],
                                   preferred_element_type=jnp.float32)

    barrier_sem = pltpu.get_barrier_semaphore()
    for axis in _AXES:
        partner, _ = _partner_and_bit(my, axis)
        pl.semaphore_signal(barrier_sem, inc=1, device_id=(partner,),
                            device_id_type=pl.DeviceIdType.MESH)
    pl.semaphore_wait(barrier_sem, N_STEPS)

    lo = [jnp.int32(st * SROWS) for st in range(N_STREAMS)]
    n = [SROWS] * N_STREAMS
    for step in range(N_STEPS):
        rdmas = []
        for st in range(N_STREAMS):
            partner, bit = _partner_and_bit(my, _ORDERS[st][step])
            half = n[st] // 2
            keep_lo = lo[st] + bit * half
            send_lo = lo[st] + (1 - bit) * half
            rdma = pltpu.make_async_remote_copy(
                src_ref=kv_ref.at[pl.ds(send_lo, half), :],
                dst_ref=slots[step].at[st],
                send_sem=rs_send.at[step, st],
                recv_sem=rs_recv.at[step, st],
                device_id=(partner,),
                device_id_type=pl.DeviceIdType.MESH,
            )
            rdma.start()
            rdmas.append(rdma)
            lo[st] = keep_lo
            n[st] = half
        for st in range(N_STREAMS):
            rdmas[st].wait()
            kv_ref[pl.ds(lo[st], n[st]), :] = (
                kv_ref[pl.ds(lo[st], n[st]), :] + slots[step][st]
            )

    for step in reversed(range(N_STEPS)):
        rdmas = []
        for st in range(N_STREAMS):
            partner, bit = _partner_and_bit(my, _ORDERS[st][step])
            rdma = pltpu.make_async_remote_copy(
                src_ref=kv_ref.at[pl.ds(lo[st], n[st]), :],
                dst_ref=kv_ref.at[pl.ds(lo[st], n[st]), :],
                send_sem=ag_send.at[step, st],
                recv_sem=ag_recv.at[step, st],
                device_id=(partner,),
                device_id_type=pl.DeviceIdType.MESH,
            )
            rdma.start()
            rdmas.append(rdma)
            lo[st] = lo[st] - bit * n[st]
            n[st] = 2 * n[st]
        for st in range(N_STREAMS):
            rdmas[st].wait()


def _allreduce_kv(x, Wdkv, Wuk, Wuv):
    return pl.pallas_call(
        _allreduce_kv_body,
        out_shape=jax.ShapeDtypeStruct((ROWS, KV_COLS), jnp.float32),
        in_specs=[pl.BlockSpec(memory_space=pltpu.VMEM)] * 4,
        out_specs=pl.BlockSpec(memory_space=pltpu.VMEM),
        scratch_shapes=(
            [pltpu.VMEM((N_STREAMS, SROWS >> (s + 1), KV_COLS), jnp.float32)
             for s in range(N_STEPS)]
            + [pltpu.SemaphoreType.DMA((N_STEPS, N_STREAMS))] * 4
        ),
        compiler_params=pltpu.CompilerParams(collective_id=0),
    )(x, Wdkv, Wuk, Wuv)


def kernel(x, Wdkv, Wuk, Wuv, Wq, Wqr, Wkr, Wo):
    kv = _allreduce_kv(x, Wdkv, Wuk, Wuv)
    K = kv[:, :D].reshape(B, S, H, Dh)
    V = kv[:, D:].reshape(B, S, H, Dh)
    x2 = x.reshape(ROWS, D)
    Q = (x2 @ Wq).reshape(B, S, H, Dh)
    Qr = (x2 @ Wqr).reshape(B, S, H, Dr)
    Kr = (x2 @ Wkr).reshape(B, S, Dr)
    scale = (Dh + Dr) ** -0.5
    scores = (jnp.einsum("bshd,bthd->bhst", Q, K)
              + jnp.einsum("bshd,btd->bhst", Qr, Kr)) * scale
    m = scores.max(-1, keepdims=True)
    p = jnp.exp(scores - m)
    p = p / p.sum(-1, keepdims=True)
    O = jnp.einsum("bhst,bthd->bshd", p, V).reshape(ROWS, H * Dh)
    return (O @ Wo).reshape(B, S, D)
